# baseline (speedup 1.0000x reference)
import jax
import jax.numpy as jnp
from jax.experimental import pallas as pl
from jax.experimental.pallas import tpu as pltpu
from jax.experimental.pallas import tpu_sc as plsc

N = 64
C = 1024
H = 14
W = 14
HW = H * W
K = 8
COUT = 256
P = 49
NSTEP = HW // P


def _pass1_body(x_ref, rows_ref, cols_ref, ind_ref, fsum_scr, cnt_scr):
    i = pl.program_id(0)
    fb = jnp.sum(x_ref[...], axis=2)
    fsum_scr[i] = fb

    ii = jax.lax.broadcasted_iota(jnp.int32, (P, P, N), 0)
    jj = jax.lax.broadcasted_iota(jnp.int32, (P, P, N), 1)
    self_gt = (fb[:, None, :] > fb[None, :, :]).astype(jnp.int32)
    self_tie = ((fb[:, None, :] == fb[None, :, :]) & (ii < jj)).astype(
        jnp.int32)
    cnt_scr[i] = jnp.sum(self_gt + self_tie, axis=0)

    for t in range(NSTEP - 1):
        @pl.when(t < i)
        def _cross():
            fa = fsum_scr[t]
            gt = (fa[:, None, :] > fb[None, :, :]).astype(jnp.int32)
            eq = (fa[:, None, :] == fb[None, :, :]).astype(jnp.int32)
            cnt_scr[i] += jnp.sum(gt + eq, axis=0)
            cnt_scr[t] += P - jnp.sum(gt + eq, axis=1)

    @pl.when(i == NSTEP - 1)
    def _select():
        rank = jnp.transpose(cnt_scr[...].reshape(HW, N))

        kk = jax.lax.broadcasted_iota(jnp.int32, (N, 16, HW), 1)
        targets = jnp.where(kk < K, kk, kk + (HW // 2 - K // 2 - 1 - K))
        onehot = (rank[:, None, :] == targets)
        pp = jax.lax.broadcasted_iota(jnp.int32, (N, 16, HW), 2)
        idx = jnp.sum(jnp.where(onehot, pp, 0), axis=2)

        rows_ref[...] = idx // W
        cols_ref[...] = idx - (idx // W) * W
        nn = jax.lax.broadcasted_iota(jnp.int32, (N, 16), 0)
        lin = idx * N + nn
        ind_ref[...] = jnp.concatenate(
            [lin[:, :K].reshape(4, 128), lin[:, K:].reshape(4, 128)], axis=0)


def _sc_gather(x2d, ind):
    nw = 2 * 16
    b_per_w = 2 * N * K // nw
    mesh = plsc.VectorSubcoreMesh(core_axis_name="c", subcore_axis_name="s")

    @pl.kernel(out_type=jax.ShapeDtypeStruct((2 * N * K, C), jnp.float32),
               mesh=mesh,
               scratch_types=[
                   pltpu.VMEM((b_per_w,), jnp.int32),
                   pltpu.VMEM((b_per_w, C), jnp.float32),
                   pltpu.SemaphoreType.DMA,
               ])
    def gather_kernel(x_hbm, i_hbm, o_hbm, idx_v, rows_v, sem):
        wid = jax.lax.axis_index("s") * 2 + jax.lax.axis_index("c")
        base = wid * b_per_w
        pltpu.sync_copy(i_hbm.at[pl.ds(base, b_per_w)], idx_v)
        pltpu.async_copy(x_hbm.at[idx_v], rows_v, sem).wait()
        pltpu.sync_copy(rows_v, o_hbm.at[pl.ds(base, b_per_w)])

    return gather_kernel(x2d, ind)


def _pass3_body(g_ref, wmax_ref, wmed_ref,
                gmax_g_ref, gmax_b_ref, gmed_g_ref, gmed_b_ref,
                rows_ref, cols_ref, ymax_ref, ymed_ref):
    rows = rows_ref[...].astype(jnp.float32)
    cols = cols_ref[...].astype(jnp.float32)

    def branch(g, w_ref, gamma_ref, beta_ref, sl, y_ref):
        x = jax.lax.dot_general(
            g, w_ref[...],
            dimension_numbers=(((1,), (1,)), ((), ())),
            preferred_element_type=jnp.float32)
        mean = jnp.mean(x, axis=0, keepdims=True)
        var = jnp.mean((x - mean) ** 2, axis=0, keepdims=True)
        x = (x - mean) / jnp.sqrt(var + 1e-5) * gamma_ref[...] + beta_ref[...]
        x = jnp.maximum(x, 0.0)
        x = x.reshape(N, K, COUT)

        r = rows[:, sl:sl + K]
        c = cols[:, sl:sl + K]
        dr = r[:, :, None] - r[:, None, :]
        dc = c[:, :, None] - c[:, None, :]
        d = jnp.sqrt(dr * dr + dc * dc)
        dmax = jnp.max(d, axis=(1, 2), keepdims=True)
        a = jnp.exp(-d / (dmax + 1e-6))
        i1 = jax.lax.broadcasted_iota(jnp.int32, (N, K, K), 1)
        i2 = jax.lax.broadcasted_iota(jnp.int32, (N, K, K), 2)
        a = a + (i1 == i2).astype(jnp.float32)
        deg = jnp.sum(a, axis=2)
        dinv = 1.0 / jnp.sqrt(deg + 1e-6)
        lap = a * dinv[:, :, None] * dinv[:, None, :]

        y = jnp.zeros((N, K, COUT), jnp.float32)
        for j in range(K):
            y = y + lap[:, :, j][:, :, None] * x[:, j, :][:, None, :]
        for k in range(K):
            y_ref[:, k * COUT:(k + 1) * COUT] = y[:, k, :]

    branch(g_ref[0:N * K, :], wmax_ref, gmax_g_ref, gmax_b_ref, 0, ymax_ref)
    branch(g_ref[N * K:2 * N * K, :], wmed_ref, gmed_g_ref, gmed_b_ref, K,
           ymed_ref)


def kernel(fpam_output, resnet_output, conv_max_w, conv_med_w,
           bn_max_gamma, bn_max_beta, bn_med_gamma, bn_med_beta):
    del resnet_output
    xt = jnp.transpose(fpam_output, (2, 3, 0, 1)).reshape(HW, N, C)

    rows, cols, ind2d = pl.pallas_call(
        _pass1_body,
        grid=(NSTEP,),
        in_specs=[pl.BlockSpec((P, N, C), lambda i: (i, 0, 0))],
        out_specs=[
            pl.BlockSpec((N, 16), lambda i: (0, 0)),
            pl.BlockSpec((N, 16), lambda i: (0, 0)),
            pl.BlockSpec((8, 128), lambda i: (0, 0)),
        ],
        out_shape=[
            jax.ShapeDtypeStruct((N, 16), jnp.int32),
            jax.ShapeDtypeStruct((N, 16), jnp.int32),
            jax.ShapeDtypeStruct((8, 128), jnp.int32),
        ],
        scratch_shapes=[pltpu.VMEM((NSTEP, P, N), jnp.float32),
                        pltpu.VMEM((NSTEP, P, N), jnp.int32)],
    )(xt)

    g = _sc_gather(xt.reshape(HW * N, C), ind2d.reshape(2 * N * K))

    ymax, ymed = pl.pallas_call(
        _pass3_body,
        out_shape=[
            jax.ShapeDtypeStruct((N, K * COUT), jnp.float32),
            jax.ShapeDtypeStruct((N, K * COUT), jnp.float32),
        ],
    )(g, conv_max_w, conv_med_w,
      bn_max_gamma.reshape(1, COUT), bn_max_beta.reshape(1, COUT),
      bn_med_gamma.reshape(1, COUT), bn_med_beta.reshape(1, COUT),
      rows, cols)

    return (ymax, ymed, rows, cols)


# speedup vs baseline: 2.8465x; 2.8465x over previous
import jax
import jax.numpy as jnp
from jax.experimental import pallas as pl
from jax.experimental.pallas import tpu as pltpu
from jax.experimental.pallas import tpu_sc as plsc

N = 64
C = 1024
H = 14
W = 14
HW = H * W
K = 8
COUT = 256
P = 49
NSTEP = HW // P


def _pass1_body(x_ref, rows_ref, cols_ref, ind_ref, fsum_scr):
    i = pl.program_id(0)
    fsum_scr[i] = jnp.sum(x_ref[...], axis=2)

    @pl.when(i == NSTEP - 1)
    def _select():
        fs = jnp.transpose(fsum_scr[...].reshape(HW, N))
        fi = fs[:, :, None]
        fj = fs[:, None, :]
        ii = jax.lax.broadcasted_iota(jnp.int32, (N, HW, HW), 1)
        jj = jax.lax.broadcasted_iota(jnp.int32, (N, HW, HW), 2)
        gt = (fi > fj).astype(jnp.int32)
        tie = ((fi == fj) & (ii < jj)).astype(jnp.int32)
        rank = jnp.sum(gt + tie, axis=1)

        kk = jax.lax.broadcasted_iota(jnp.int32, (N, 16, HW), 1)
        targets = jnp.where(kk < K, kk, kk + (HW // 2 - K // 2 - 1 - K))
        onehot = (rank[:, None, :] == targets)
        pp = jax.lax.broadcasted_iota(jnp.int32, (N, 16, HW), 2)
        idx = jnp.sum(jnp.where(onehot, pp, 0), axis=2)

        rows_ref[...] = idx // W
        cols_ref[...] = idx - (idx // W) * W
        nn = jax.lax.broadcasted_iota(jnp.int32, (N, 16), 0)
        lin = idx * N + nn
        ind_ref[...] = jnp.concatenate(
            [lin[:, :K].reshape(4, 128), lin[:, K:].reshape(4, 128)], axis=0)


def _sc_gather(x2d, ind):
    nw = 2 * 16
    b_per_w = 2 * N * K // nw
    mesh = plsc.VectorSubcoreMesh(core_axis_name="c", subcore_axis_name="s")

    @pl.kernel(out_type=jax.ShapeDtypeStruct((2 * N * K, C), jnp.float32),
               mesh=mesh,
               scratch_types=[
                   pltpu.VMEM((b_per_w,), jnp.int32),
                   pltpu.VMEM((b_per_w, C), jnp.float32),
                   pltpu.SemaphoreType.DMA,
               ])
    def gather_kernel(x_hbm, i_hbm, o_hbm, idx_v, rows_v, sem):
        wid = jax.lax.axis_index("s") * 2 + jax.lax.axis_index("c")
        base = wid * b_per_w
        pltpu.sync_copy(i_hbm.at[pl.ds(base, b_per_w)], idx_v)
        pltpu.async_copy(x_hbm.at[idx_v], rows_v, sem).wait()
        pltpu.sync_copy(rows_v, o_hbm.at[pl.ds(base, b_per_w)])

    return gather_kernel(x2d, ind)


def _pass3_body(g_ref, wmax_ref, wmed_ref,
                gmax_g_ref, gmax_b_ref, gmed_g_ref, gmed_b_ref,
                rows_ref, cols_ref, ymax_ref, ymed_ref):
    rows = rows_ref[...].astype(jnp.float32)
    cols = cols_ref[...].astype(jnp.float32)

    def branch(g, w_ref, gamma_ref, beta_ref, sl, y_ref):
        x = jax.lax.dot_general(
            g, w_ref[...],
            dimension_numbers=(((1,), (1,)), ((), ())),
            preferred_element_type=jnp.float32)
        mean = jnp.mean(x, axis=0, keepdims=True)
        var = jnp.mean((x - mean) ** 2, axis=0, keepdims=True)
        x = (x - mean) / jnp.sqrt(var + 1e-5) * gamma_ref[...] + beta_ref[...]
        x = jnp.maximum(x, 0.0)
        x = x.reshape(N, K, COUT)

        r = rows[:, sl:sl + K]
        c = cols[:, sl:sl + K]
        dr = r[:, :, None] - r[:, None, :]
        dc = c[:, :, None] - c[:, None, :]
        d = jnp.sqrt(dr * dr + dc * dc)
        dmax = jnp.max(d, axis=(1, 2), keepdims=True)
        a = jnp.exp(-d / (dmax + 1e-6))
        i1 = jax.lax.broadcasted_iota(jnp.int32, (N, K, K), 1)
        i2 = jax.lax.broadcasted_iota(jnp.int32, (N, K, K), 2)
        a = a + (i1 == i2).astype(jnp.float32)
        deg = jnp.sum(a, axis=2)
        dinv = 1.0 / jnp.sqrt(deg + 1e-6)
        lap = a * dinv[:, :, None] * dinv[:, None, :]

        y = jnp.zeros((N, K, COUT), jnp.float32)
        for j in range(K):
            y = y + lap[:, :, j][:, :, None] * x[:, j, :][:, None, :]
        for k in range(K):
            y_ref[:, k * COUT:(k + 1) * COUT] = y[:, k, :]

    branch(g_ref[0:N * K, :], wmax_ref, gmax_g_ref, gmax_b_ref, 0, ymax_ref)
    branch(g_ref[N * K:2 * N * K, :], wmed_ref, gmed_g_ref, gmed_b_ref, K,
           ymed_ref)


def kernel(fpam_output, resnet_output, conv_max_w, conv_med_w,
           bn_max_gamma, bn_max_beta, bn_med_gamma, bn_med_beta):
    del resnet_output
    xt = jnp.transpose(fpam_output, (2, 3, 0, 1)).reshape(HW, N, C)

    rows, cols, ind2d = pl.pallas_call(
        _pass1_body,
        grid=(NSTEP,),
        in_specs=[pl.BlockSpec((P, N, C), lambda i: (i, 0, 0))],
        out_specs=[
            pl.BlockSpec((N, 16), lambda i: (0, 0)),
            pl.BlockSpec((N, 16), lambda i: (0, 0)),
            pl.BlockSpec((8, 128), lambda i: (0, 0)),
        ],
        out_shape=[
            jax.ShapeDtypeStruct((N, 16), jnp.int32),
            jax.ShapeDtypeStruct((N, 16), jnp.int32),
            jax.ShapeDtypeStruct((8, 128), jnp.int32),
        ],
        scratch_shapes=[pltpu.VMEM((NSTEP, P, N), jnp.float32)],
    )(xt)

    g = _sc_gather(xt.reshape(HW * N, C), ind2d.reshape(2 * N * K))

    ymax, ymed = pl.pallas_call(
        _pass3_body,
        out_shape=[
            jax.ShapeDtypeStruct((N, K * COUT), jnp.float32),
            jax.ShapeDtypeStruct((N, K * COUT), jnp.float32),
        ],
    )(g, conv_max_w, conv_med_w,
      bn_max_gamma.reshape(1, COUT), bn_max_beta.reshape(1, COUT),
      bn_med_gamma.reshape(1, COUT), bn_med_beta.reshape(1, COUT),
      rows, cols)

    return (ymax, ymed, rows, cols)


# speedup vs baseline: 2.8548x; 1.0029x over previous
import jax
import jax.numpy as jnp
from jax.experimental import pallas as pl
from jax.experimental.pallas import tpu as pltpu
from jax.experimental.pallas import tpu_sc as plsc

N = 64
C = 1024
H = 14
W = 14
HW = H * W
K = 8
COUT = 256
P = 49
NSTEP = HW // P


def _pass1_body(x_ref, rows_ref, cols_ref, ind_ref, fsum_scr):
    i = pl.program_id(0)
    fsum_scr[i] = jnp.sum(x_ref[...], axis=2)

    @pl.when(i == NSTEP - 1)
    def _select():
        fs = jnp.transpose(fsum_scr[...].reshape(HW, N))
        fi = fs[:, :, None]
        fj = fs[:, None, :]
        ii = jax.lax.broadcasted_iota(jnp.int32, (N, HW, HW), 1)
        jj = jax.lax.broadcasted_iota(jnp.int32, (N, HW, HW), 2)
        before = (fi > fj) | ((fi == fj) & (ii < jj))
        rank = jnp.sum(before.astype(jnp.int32), axis=1)

        kk = jax.lax.broadcasted_iota(jnp.int32, (N, 16, HW), 1)
        targets = jnp.where(kk < K, kk, kk + (HW // 2 - K // 2 - 1 - K))
        onehot = (rank[:, None, :] == targets)
        pp = jax.lax.broadcasted_iota(jnp.int32, (N, 16, HW), 2)
        idx = jnp.sum(jnp.where(onehot, pp, 0), axis=2)

        rows_ref[...] = idx // W
        cols_ref[...] = idx - (idx // W) * W
        nn = jax.lax.broadcasted_iota(jnp.int32, (N, 16), 0)
        lin = idx * N + nn
        ind_ref[...] = jnp.concatenate(
            [lin[:, :K].reshape(4, 128), lin[:, K:].reshape(4, 128)], axis=0)


def _sc_gather(x2d, ind):
    nw = 2 * 16
    b_per_w = 2 * N * K // nw
    mesh = plsc.VectorSubcoreMesh(core_axis_name="c", subcore_axis_name="s")

    @pl.kernel(out_type=jax.ShapeDtypeStruct((2 * N * K, C), jnp.float32),
               mesh=mesh,
               scratch_types=[
                   pltpu.VMEM((b_per_w,), jnp.int32),
                   pltpu.VMEM((b_per_w, C), jnp.float32),
                   pltpu.SemaphoreType.DMA,
               ])
    def gather_kernel(x_hbm, i_hbm, o_hbm, idx_v, rows_v, sem):
        wid = jax.lax.axis_index("s") * 2 + jax.lax.axis_index("c")
        base = wid * b_per_w
        pltpu.sync_copy(i_hbm.at[pl.ds(base, b_per_w)], idx_v)
        pltpu.async_copy(x_hbm.at[idx_v], rows_v, sem).wait()
        pltpu.sync_copy(rows_v, o_hbm.at[pl.ds(base, b_per_w)])

    return gather_kernel(x2d, ind)


def _pass3_body(g_ref, wmax_ref, wmed_ref,
                gmax_g_ref, gmax_b_ref, gmed_g_ref, gmed_b_ref,
                rows_ref, cols_ref, ymax_ref, ymed_ref):
    rows = rows_ref[...].astype(jnp.float32)
    cols = cols_ref[...].astype(jnp.float32)

    def branch(g, w_ref, gamma_ref, beta_ref, sl, y_ref):
        x = jax.lax.dot_general(
            g, w_ref[...],
            dimension_numbers=(((1,), (1,)), ((), ())),
            preferred_element_type=jnp.float32)
        mean = jnp.mean(x, axis=0, keepdims=True)
        var = jnp.mean((x - mean) ** 2, axis=0, keepdims=True)
        x = (x - mean) / jnp.sqrt(var + 1e-5) * gamma_ref[...] + beta_ref[...]
        x = jnp.maximum(x, 0.0)
        x = x.reshape(N, K, COUT)

        r = rows[:, sl:sl + K]
        c = cols[:, sl:sl + K]
        dr = r[:, :, None] - r[:, None, :]
        dc = c[:, :, None] - c[:, None, :]
        d = jnp.sqrt(dr * dr + dc * dc)
        dmax = jnp.max(d, axis=(1, 2), keepdims=True)
        a = jnp.exp(-d / (dmax + 1e-6))
        i1 = jax.lax.broadcasted_iota(jnp.int32, (N, K, K), 1)
        i2 = jax.lax.broadcasted_iota(jnp.int32, (N, K, K), 2)
        a = a + (i1 == i2).astype(jnp.float32)
        deg = jnp.sum(a, axis=2)
        dinv = 1.0 / jnp.sqrt(deg + 1e-6)
        lap = a * dinv[:, :, None] * dinv[:, None, :]

        y = jnp.zeros((N, K, COUT), jnp.float32)
        for j in range(K):
            y = y + lap[:, :, j][:, :, None] * x[:, j, :][:, None, :]
        for k in range(K):
            y_ref[:, k * COUT:(k + 1) * COUT] = y[:, k, :]

    branch(g_ref[0:N * K, :], wmax_ref, gmax_g_ref, gmax_b_ref, 0, ymax_ref)
    branch(g_ref[N * K:2 * N * K, :], wmed_ref, gmed_g_ref, gmed_b_ref, K,
           ymed_ref)


def kernel(fpam_output, resnet_output, conv_max_w, conv_med_w,
           bn_max_gamma, bn_max_beta, bn_med_gamma, bn_med_beta):
    del resnet_output
    xt = jnp.transpose(fpam_output, (2, 3, 0, 1)).reshape(HW, N, C)

    rows, cols, ind2d = pl.pallas_call(
        _pass1_body,
        grid=(NSTEP,),
        in_specs=[pl.BlockSpec((P, N, C), lambda i: (i, 0, 0))],
        out_specs=[
            pl.BlockSpec((N, 16), lambda i: (0, 0)),
            pl.BlockSpec((N, 16), lambda i: (0, 0)),
            pl.BlockSpec((8, 128), lambda i: (0, 0)),
        ],
        out_shape=[
            jax.ShapeDtypeStruct((N, 16), jnp.int32),
            jax.ShapeDtypeStruct((N, 16), jnp.int32),
            jax.ShapeDtypeStruct((8, 128), jnp.int32),
        ],
        scratch_shapes=[pltpu.VMEM((NSTEP, P, N), jnp.float32)],
    )(xt)

    g = _sc_gather(xt.reshape(HW * N, C), ind2d.reshape(2 * N * K))

    ymax, ymed = pl.pallas_call(
        _pass3_body,
        out_shape=[
            jax.ShapeDtypeStruct((N, K * COUT), jnp.float32),
            jax.ShapeDtypeStruct((N, K * COUT), jnp.float32),
        ],
    )(g, conv_max_w, conv_med_w,
      bn_max_gamma.reshape(1, COUT), bn_max_beta.reshape(1, COUT),
      bn_med_gamma.reshape(1, COUT), bn_med_beta.reshape(1, COUT),
      rows, cols)

    return (ymax, ymed, rows, cols)
